# baseline (speedup 1.0000x reference)
import functools

import jax
import jax.numpy as jnp
from jax import lax
from jax.experimental import pallas as pl
from jax.experimental.pallas import tpu as pltpu
from jax.experimental.pallas import tpu_sc as plsc

_CHUNK = 128


@functools.cache
def _build(V, D, B):
    info = plsc.get_sparse_core_info()
    NC, NS = info.num_cores, info.num_subcores
    NW = NC * NS
    b_per_w = B // NW
    n_ch = b_per_w // _CHUNK
    stage_rows = 128
    mesh = plsc.VectorSubcoreMesh(core_axis_name="c", subcore_axis_name="s")

    @functools.partial(
        pl.kernel,
        mesh=mesh,
        out_type=jax.ShapeDtypeStruct((B, D), jnp.float32),
        scratch_types=[
            pltpu.VMEM_SHARED((V, D), jnp.float32),
            pltpu.VMEM((n_ch, _CHUNK), jnp.int32),
            pltpu.VMEM((n_ch, _CHUNK, D), jnp.float32),
            pltpu.SemaphoreType.DMA,
            pltpu.SemaphoreType.DMA,
            pltpu.SemaphoreType.DMA,
        ],
    )
    def k(table_hbm, idx_hbm, out_hbm, tab_s, idx_v, rows_v, tsem, gsem, osem):
        c = lax.axis_index("c")
        s = lax.axis_index("s")
        wid = s * NC + c

        n_stage = (V + stage_rows - 1) // stage_rows
        for i in range(n_stage):
            r0 = i * stage_rows
            nrows = min(stage_rows, V - r0)

            @pl.when(s == i)
            def _stage_table(r0=r0, nrows=nrows):
                pltpu.async_copy(
                    table_hbm.at[pl.ds(r0, nrows)],
                    tab_s.at[pl.ds(r0, nrows)],
                    tsem,
                ).wait()

        pltpu.sync_copy(idx_hbm.at[wid], idx_v)
        plsc.subcore_barrier()

        gathers = []
        for j in range(n_ch):
            gathers.append(
                pltpu.async_copy(tab_s.at[idx_v.at[j]], rows_v.at[j], gsem)
            )
        for g in gathers:
            g.wait()
        pltpu.async_copy(
            rows_v.reshape(b_per_w, D),
            out_hbm.at[pl.ds(wid * b_per_w, b_per_w)],
            osem,
        ).wait()

    return k, NW, n_ch


def kernel(class_number, embedding):
    V, D = embedding.shape
    B = class_number.shape[0]
    k, NW, n_ch = _build(V, D, B)
    idx = class_number.astype(jnp.int32).reshape(NW, n_ch, _CHUNK)
    return k(embedding, idx)


# speedup vs baseline: 1.0451x; 1.0451x over previous
import functools

import jax
import jax.numpy as jnp
from jax import lax
from jax.experimental import pallas as pl
from jax.experimental.pallas import tpu as pltpu
from jax.experimental.pallas import tpu_sc as plsc

_CHUNK = 128


@functools.cache
def _build(V, D, B):
    info = plsc.get_sparse_core_info()
    NC, NS = info.num_cores, info.num_subcores
    NW = NC * NS
    b_per_w = B // NW
    n_ch = b_per_w // _CHUNK
    stage_rows = 128
    mesh = plsc.VectorSubcoreMesh(core_axis_name="c", subcore_axis_name="s")

    @functools.partial(
        pl.kernel,
        mesh=mesh,
        out_type=jax.ShapeDtypeStruct((B, D), jnp.float32),
        scratch_types=[
            pltpu.VMEM_SHARED((V, D), jnp.float32),
            pltpu.VMEM((n_ch, _CHUNK), jnp.int32),
            pltpu.VMEM((n_ch, _CHUNK, D), jnp.float32),
            pltpu.SemaphoreType.DMA,
            pltpu.SemaphoreType.DMA,
            pltpu.SemaphoreType.DMA,
        ],
    )
    def k(table_hbm, idx_hbm, out_hbm, tab_s, idx_v, rows_v, tsem, gsem, osem):
        c = lax.axis_index("c")
        s = lax.axis_index("s")
        wid = s * NC + c

        pltpu.sync_copy(idx_hbm.at[wid], idx_v)
        gathers = [
            pltpu.async_copy(table_hbm.at[idx_v.at[0]], rows_v.at[0], gsem)
        ]

        n_stage = (V + stage_rows - 1) // stage_rows
        for i in range(n_stage):
            r0 = i * stage_rows
            nrows = min(stage_rows, V - r0)

            @pl.when(s == i)
            def _stage_table(r0=r0, nrows=nrows):
                pltpu.async_copy(
                    table_hbm.at[pl.ds(r0, nrows)],
                    tab_s.at[pl.ds(r0, nrows)],
                    tsem,
                ).wait()

        plsc.subcore_barrier()

        for j in range(1, n_ch):
            gathers.append(
                pltpu.async_copy(tab_s.at[idx_v.at[j]], rows_v.at[j], gsem)
            )
        outs = []
        for j in range(n_ch):
            gathers[j].wait()
            outs.append(
                pltpu.async_copy(
                    rows_v.at[j],
                    out_hbm.at[pl.ds(wid * b_per_w + j * _CHUNK, _CHUNK)],
                    osem,
                )
            )
        for o in outs:
            o.wait()

    return k, NW, n_ch


def kernel(class_number, embedding):
    V, D = embedding.shape
    B = class_number.shape[0]
    k, NW, n_ch = _build(V, D, B)
    idx = class_number.astype(jnp.int32).reshape(NW, n_ch, _CHUNK)
    return k(embedding, idx)


# speedup vs baseline: 1.0489x; 1.0037x over previous
import functools

import jax
import jax.numpy as jnp
from jax import lax
from jax.experimental import pallas as pl
from jax.experimental.pallas import tpu as pltpu
from jax.experimental.pallas import tpu_sc as plsc

_CHUNK = 128


@functools.cache
def _build(V, D, B):
    info = plsc.get_sparse_core_info()
    NC, NS = info.num_cores, info.num_subcores
    NW = NC * NS
    b_per_w = B // NW
    n_ch = b_per_w // _CHUNK
    stage_rows = 128
    mesh = plsc.VectorSubcoreMesh(core_axis_name="c", subcore_axis_name="s")

    @functools.partial(
        pl.kernel,
        mesh=mesh,
        out_type=jax.ShapeDtypeStruct((B, D), jnp.float32),
        scratch_types=[
            pltpu.VMEM_SHARED((V, D), jnp.float32),
            pltpu.VMEM((n_ch, _CHUNK), jnp.int32),
            pltpu.VMEM((n_ch, _CHUNK, D), jnp.float32),
            pltpu.SemaphoreType.DMA,
            pltpu.SemaphoreType.DMA,
            pltpu.SemaphoreType.DMA,
        ],
    )
    def k(table_hbm, idx_hbm, out_hbm, tab_s, idx_v, rows_v, tsem, gsem, osem):
        c = lax.axis_index("c")
        s = lax.axis_index("s")
        wid = s * NC + c

        n_stage = (V + stage_rows - 1) // stage_rows
        for i in range(n_stage):
            r0 = i * stage_rows
            nrows = min(stage_rows, V - r0)

            @pl.when(s == i)
            def _stage_table(r0=r0, nrows=nrows):
                pltpu.async_copy(
                    table_hbm.at[pl.ds(r0, nrows)],
                    tab_s.at[pl.ds(r0, nrows)],
                    tsem,
                ).wait()

        pltpu.sync_copy(idx_hbm.at[wid], idx_v)
        plsc.subcore_barrier()

        gathers = []
        for j in range(n_ch):
            gathers.append(
                pltpu.async_copy(tab_s.at[idx_v.at[j]], rows_v.at[j], gsem)
            )
        outs = []
        for j in range(n_ch):
            gathers[j].wait()
            outs.append(
                pltpu.async_copy(
                    rows_v.at[j],
                    out_hbm.at[pl.ds(wid * b_per_w + j * _CHUNK, _CHUNK)],
                    osem,
                )
            )
        for o in outs:
            o.wait()

    return k, NW, n_ch


def kernel(class_number, embedding):
    V, D = embedding.shape
    B = class_number.shape[0]
    k, NW, n_ch = _build(V, D, B)
    idx = class_number.astype(jnp.int32).reshape(NW, n_ch, _CHUNK)
    return k(embedding, idx)
